# Initial kernel scaffold; baseline (speedup 1.0000x reference)
import functools

import jax
import jax.numpy as jnp
from jax import lax
from jax.experimental import pallas as pl
from jax.experimental.pallas import tpu as pltpu
from jax.experimental.pallas import tpu_sc as plsc

_EMBED = 32
_NTOK = 8192
_TBLK = 512
_NT = _NTOK // _TBLK
_CB0, _CB1 = 10000, 100
_CB0P, _CB1P = 10240, 128
_CHUNK0 = 2048

_BIG_I32 = jnp.int32(2**30)


def _scan_codebook(cb_ref, wn_ref, embT, en, n_rows, chunk, out_ref):
    best_d = jnp.full((1, _TBLK), jnp.inf, dtype=jnp.float32)
    best_i = jnp.zeros((1, _TBLK), dtype=jnp.int32)
    for c in range(n_rows // chunk):
        cb = cb_ref[pl.ds(c * chunk, chunk), :]
        wn = wn_ref[pl.ds(c * chunk, chunk), :]
        mm = jnp.dot(cb, embT, preferred_element_type=jnp.float32)
        dist = (wn + en) - 2.0 * mm
        cmin = jnp.min(dist, axis=0, keepdims=True)
        iota = lax.broadcasted_iota(jnp.int32, (chunk, _TBLK), 0)
        cidx = jnp.min(jnp.where(dist == cmin, iota, _BIG_I32),
                       axis=0, keepdims=True)
        upd = cmin < best_d
        best_i = jnp.where(upd, cidx + c * chunk, best_i)
        best_d = jnp.where(upd, cmin, best_d)
    out_ref[...] = best_i.reshape(1, 1, _TBLK)


def _argmin_body(embT_ref, en_ref, cb0_ref, wn0_ref, cb1_ref, wn1_ref,
                 idx0_ref, idx1_ref):
    embT = embT_ref[...]
    en = en_ref[0]
    _scan_codebook(cb0_ref, wn0_ref, embT, en, _CB0P, _CHUNK0, idx0_ref)
    _scan_codebook(cb1_ref, wn1_ref, embT, en, _CB1P, _CB1P, idx1_ref)


def _argmin_indices(embT, en3, cb0p, wn0p, cb1p, wn1p):
    return pl.pallas_call(
        _argmin_body,
        grid=(_NT,),
        in_specs=[
            pl.BlockSpec((_EMBED, _TBLK), lambda i: (0, i)),
            pl.BlockSpec((1, 1, _TBLK), lambda i: (i, 0, 0)),
            pl.BlockSpec((_CB0P, _EMBED), lambda i: (0, 0)),
            pl.BlockSpec((_CB0P, 1), lambda i: (0, 0)),
            pl.BlockSpec((_CB1P, _EMBED), lambda i: (0, 0)),
            pl.BlockSpec((_CB1P, 1), lambda i: (0, 0)),
        ],
        out_specs=[
            pl.BlockSpec((1, 1, _TBLK), lambda i: (i, 0, 0)),
            pl.BlockSpec((1, 1, _TBLK), lambda i: (i, 0, 0)),
        ],
        out_shape=[
            jax.ShapeDtypeStruct((_NT, 1, _TBLK), jnp.int32),
            jax.ShapeDtypeStruct((_NT, 1, _TBLK), jnp.int32),
        ],
    )(embT, en3, cb0p, wn0p, cb1p, wn1p)


_GCH = 128


def _gather_rows(cb0, cb1, idx0, idx1):
    info = plsc.get_sparse_core_info()
    nc, ns = info.num_cores, info.num_subcores
    nwork = nc * ns
    rows_w = _NTOK // nwork
    mesh = plsc.VectorSubcoreMesh(core_axis_name="c", subcore_axis_name="s")

    @functools.partial(
        pl.kernel,
        mesh=mesh,
        out_type=[jax.ShapeDtypeStruct((_NTOK, _EMBED), jnp.float32),
                  jax.ShapeDtypeStruct((_NTOK, _EMBED), jnp.float32)],
        scratch_types=[
            pltpu.VMEM((_GCH,), jnp.int32),
            pltpu.VMEM((_GCH, _EMBED), jnp.float32),
            pltpu.SemaphoreType.DMA,
        ],
    )
    def k(cb0_hbm, cb1_hbm, idx0_hbm, idx1_hbm, out0_hbm, out1_hbm,
          idx_v, rows_v, sem):
        wid = lax.axis_index("s") * nc + lax.axis_index("c")
        base = wid * rows_w
        for tbl, idx_hbm, out_hbm in ((cb0_hbm, idx0_hbm, out0_hbm),
                                      (cb1_hbm, idx1_hbm, out1_hbm)):
            for p in range(rows_w // _GCH):
                b = base + p * _GCH
                pltpu.sync_copy(idx_hbm.at[pl.ds(b, _GCH)], idx_v)
                pltpu.async_copy(tbl.at[idx_v], rows_v, sem).wait()
                pltpu.sync_copy(rows_v, out_hbm.at[pl.ds(b, _GCH)])

    return k(cb0, cb1, idx0, idx1)


def kernel(embeds, cb0, cb1):
    embT = embeds.T
    en3 = jnp.sum(embeds ** 2, axis=1).reshape(_NT, 1, _TBLK)
    wn0 = jnp.sum(cb0 ** 2, axis=1)
    wn1 = jnp.sum(cb1 ** 2, axis=1)
    inf = jnp.float32(jnp.inf)
    wn0p = jnp.concatenate(
        [wn0, jnp.full((_CB0P - _CB0,), inf, jnp.float32)]).reshape(_CB0P, 1)
    wn1p = jnp.concatenate(
        [wn1, jnp.full((_CB1P - _CB1,), inf, jnp.float32)]).reshape(_CB1P, 1)
    cb0p = jnp.pad(cb0, ((0, _CB0P - _CB0), (0, 0)))
    cb1p = jnp.pad(cb1, ((0, _CB1P - _CB1), (0, 0)))
    idx0_3d, idx1_3d = _argmin_indices(embT, en3, cb0p, wn0p, cb1p, wn1p)
    q0, q1 = _gather_rows(cb0, cb1, idx0_3d.reshape(-1), idx1_3d.reshape(-1))
    return jnp.stack([q0, q1], axis=0)


# speedup vs baseline: 1.1783x; 1.1783x over previous
import functools

import jax
import jax.numpy as jnp
from jax import lax
from jax.experimental import pallas as pl
from jax.experimental.pallas import tpu as pltpu
from jax.experimental.pallas import tpu_sc as plsc

_EMBED = 32
_NTOK = 8192
_TBLK = 512
_NT = _NTOK // _TBLK
_CB0, _CB1 = 10000, 100
_CB0P, _CB1P = 10240, 128
_CHUNK0 = 2048

_BIG_I32 = 2**30


def _scan_codebook(cbT_ref, wn_ref, emb, en, n_rows, chunk, out_ref):
    best_d = jnp.full((_TBLK, 1), jnp.inf, dtype=jnp.float32)
    best_i = jnp.zeros((_TBLK, 1), dtype=jnp.int32)
    for c in range(n_rows // chunk):
        cbT = cbT_ref[:, pl.ds(c * chunk, chunk)]
        wn = wn_ref[:, pl.ds(c * chunk, chunk)]
        mm = jnp.dot(emb, cbT, preferred_element_type=jnp.float32)
        dist = (en + wn) - 2.0 * mm
        cmin = jnp.min(dist, axis=1, keepdims=True)
        iota = lax.broadcasted_iota(jnp.int32, (_TBLK, chunk), 1)
        cidx = jnp.min(jnp.where(dist == cmin, iota, _BIG_I32),
                       axis=1, keepdims=True)
        upd = cmin < best_d
        best_i = jnp.where(upd, cidx + c * chunk, best_i)
        best_d = jnp.where(upd, cmin, best_d)
    out_ref[...] = best_i


def _argmin_body(emb_ref, en_ref, cb0T_ref, wn0_ref, cb1T_ref, wn1_ref,
                 idx0_ref, idx1_ref):
    emb = emb_ref[...]
    en = en_ref[...]
    _scan_codebook(cb0T_ref, wn0_ref, emb, en, _CB0P, _CHUNK0, idx0_ref)
    _scan_codebook(cb1T_ref, wn1_ref, emb, en, _CB1P, _CB1P, idx1_ref)


def _argmin_indices(embeds, en2, cb0T, wn0, cb1T, wn1):
    return pl.pallas_call(
        _argmin_body,
        grid=(_NT,),
        in_specs=[
            pl.BlockSpec((_TBLK, _EMBED), lambda i: (i, 0)),
            pl.BlockSpec((_TBLK, 1), lambda i: (i, 0)),
            pl.BlockSpec((_EMBED, _CB0P), lambda i: (0, 0)),
            pl.BlockSpec((1, _CB0P), lambda i: (0, 0)),
            pl.BlockSpec((_EMBED, _CB1P), lambda i: (0, 0)),
            pl.BlockSpec((1, _CB1P), lambda i: (0, 0)),
        ],
        out_specs=[
            pl.BlockSpec((_TBLK, 1), lambda i: (i, 0)),
            pl.BlockSpec((_TBLK, 1), lambda i: (i, 0)),
        ],
        out_shape=[
            jax.ShapeDtypeStruct((_NTOK, 1), jnp.int32),
            jax.ShapeDtypeStruct((_NTOK, 1), jnp.int32),
        ],
    )(embeds, en2, cb0T, wn0, cb1T, wn1)


_GCH = 128


def _gather_rows(cb0, cb1, idx0, idx1):
    info = plsc.get_sparse_core_info()
    nc, ns = info.num_cores, info.num_subcores
    nwork = nc * ns
    rows_w = _NTOK // nwork
    mesh = plsc.VectorSubcoreMesh(core_axis_name="c", subcore_axis_name="s")

    @functools.partial(
        pl.kernel,
        mesh=mesh,
        out_type=[jax.ShapeDtypeStruct((_NTOK, _EMBED), jnp.float32),
                  jax.ShapeDtypeStruct((_NTOK, _EMBED), jnp.float32)],
        scratch_types=[
            pltpu.VMEM((_GCH,), jnp.int32),
            pltpu.VMEM((_GCH, _EMBED), jnp.float32),
            pltpu.SemaphoreType.DMA,
        ],
        compiler_params=pltpu.CompilerParams(use_tc_tiling_on_sc=False),
    )
    def k(cb0_hbm, cb1_hbm, idx0_hbm, idx1_hbm, out0_hbm, out1_hbm,
          idx_v, rows_v, sem):
        wid = lax.axis_index("s") * nc + lax.axis_index("c")
        base = wid * rows_w
        for tbl, idx_hbm, out_hbm in ((cb0_hbm, idx0_hbm, out0_hbm),
                                      (cb1_hbm, idx1_hbm, out1_hbm)):
            for p in range(rows_w // _GCH):
                b = base + p * _GCH
                pltpu.sync_copy(idx_hbm.at[pl.ds(b, _GCH)], idx_v)
                pltpu.async_copy(tbl.at[idx_v], rows_v, sem).wait()
                pltpu.sync_copy(rows_v, out_hbm.at[pl.ds(b, _GCH)])

    return k(cb0, cb1, idx0, idx1)


def kernel(embeds, cb0, cb1):
    en2 = jnp.sum(embeds ** 2, axis=1).reshape(_NTOK, 1)
    wn0 = jnp.sum(cb0 ** 2, axis=1)
    wn1 = jnp.sum(cb1 ** 2, axis=1)
    inf = jnp.float32(jnp.inf)
    wn0p = jnp.concatenate(
        [wn0, jnp.full((_CB0P - _CB0,), inf, jnp.float32)]).reshape(1, _CB0P)
    wn1p = jnp.concatenate(
        [wn1, jnp.full((_CB1P - _CB1,), inf, jnp.float32)]).reshape(1, _CB1P)
    cb0T = jnp.pad(cb0, ((0, _CB0P - _CB0), (0, 0))).T
    cb1T = jnp.pad(cb1, ((0, _CB1P - _CB1), (0, 0))).T
    idx0_2d, idx1_2d = _argmin_indices(embeds, en2, cb0T, wn0p, cb1T, wn1p)
    q0, q1 = _gather_rows(cb0, cb1, idx0_2d.reshape(-1), idx1_2d.reshape(-1))
    return jnp.stack([q0, q1], axis=0)
